# baseline (speedup 1.0000x reference)
import functools
import jax
import jax.numpy as jnp
from jax import lax
from jax.experimental import pallas as pl
from jax.experimental.pallas import tpu as pltpu
from jax.experimental.pallas import tpu_sc as plsc

_LANE = 16


def _make_lookup(num_entities, num_relations, embed_dim, batch):
    info = plsc.get_sparse_core_info()
    nc, ns = info.num_cores, info.num_subcores
    nw = nc * ns
    b_per_w = batch // nw
    chunk = 64
    n_chunks = b_per_w // chunk
    kd = embed_dim // _LANE

    mesh = plsc.VectorSubcoreMesh(core_axis_name="c", subcore_axis_name="s")

    @functools.partial(
        pl.kernel,
        mesh=mesh,
        out_type=[
            jax.ShapeDtypeStruct((batch, embed_dim), jnp.float32),
            jax.ShapeDtypeStruct((batch, embed_dim), jnp.float32),
        ],
        scratch_types=[
            pltpu.VMEM((b_per_w,), jnp.int32),
            pltpu.VMEM((b_per_w,), jnp.int32),
            pltpu.VMEM((chunk, embed_dim), jnp.float32),
            pltpu.VMEM((num_relations * embed_dim,), jnp.float32),
            pltpu.VMEM((chunk, embed_dim), jnp.float32),
            pltpu.SemaphoreType.DMA,
        ],
    )
    def lookup(ent_hbm, rel_hbm, eid_hbm, rid_hbm, ent_out, rel_out,
               eidx_v, ridx_v, ebuf, rel_v, rout, esem):
        wid = lax.axis_index("s") * nc + lax.axis_index("c")
        base = wid * b_per_w

        pltpu.sync_copy(eid_hbm.at[pl.ds(base, b_per_w)], eidx_v)
        pltpu.sync_copy(rid_hbm.at[pl.ds(base, b_per_w)], ridx_v)
        pltpu.sync_copy(rel_hbm, rel_v)

        def body(c, _):
            cb = c * chunk
            copies = []
            for g in range(chunk // _LANE):
                ev = eidx_v[pl.ds(cb + g * _LANE, _LANE)]
                for l in range(_LANE):
                    e = ev[l]
                    copies.append(pltpu.async_copy(
                        ent_hbm.at[pl.ds(e * embed_dim, embed_dim)],
                        ebuf.at[g * _LANE + l], esem))
            for g in range(chunk // _LANE):
                rv = ridx_v[pl.ds(cb + g * _LANE, _LANE)]
                for l in range(_LANE):
                    q = rv[l]
                    for k in range(kd):
                        rout[g * _LANE + l, pl.ds(k * _LANE, _LANE)] = (
                            rel_v[pl.ds(q * embed_dim + k * _LANE, _LANE)])
            for cp in copies:
                cp.wait()
            pltpu.sync_copy(ebuf, ent_out.at[pl.ds(base + cb, chunk)])
            pltpu.sync_copy(rout, rel_out.at[pl.ds(base + cb, chunk)])
            return _

        lax.fori_loop(0, n_chunks, body, None)

    return lookup


def kernel(entity_embedding, relation_embedding, entity_ids, relation_ids):
    num_entities, embed_dim = entity_embedding.shape
    num_relations = relation_embedding.shape[0]
    batch = entity_ids.shape[0]
    lookup = _make_lookup(num_entities, num_relations, embed_dim, batch)
    return tuple(
        lookup(
            entity_embedding.reshape(-1),
            relation_embedding.reshape(-1),
            entity_ids.astype(jnp.int32),
            relation_ids.astype(jnp.int32),
        )
    )


# speedup vs baseline: 2.8948x; 2.8948x over previous
import functools
import jax
import jax.numpy as jnp
from jax import lax
from jax.experimental import pallas as pl
from jax.experimental.pallas import tpu as pltpu
from jax.experimental.pallas import tpu_sc as plsc

_LANE = 16
_SUB = 8
_TILE = 128
_G = 4


def _make_entity_lookup(num_entities, embed_dim, batch):
    info = plsc.get_sparse_core_info()
    nc, ns = info.num_cores, info.num_subcores
    nw = nc * ns
    b_per_w = batch // nw
    tile_cols = 128
    n_tiles = b_per_w // tile_cols
    d8 = embed_dim // _SUB
    n_groups = tile_cols // _G
    n_pairs = n_groups // 2

    mesh = plsc.VectorSubcoreMesh(core_axis_name="c", subcore_axis_name="s")

    @functools.partial(
        pl.kernel,
        mesh=mesh,
        compiler_params=pltpu.CompilerParams(needs_layout_passes=False),
        out_type=jax.ShapeDtypeStruct((d8, _SUB, batch), jnp.float32),
        scratch_types=[
            pltpu.VMEM((b_per_w,), jnp.int32),
            pltpu.VMEM((_G, d8, _SUB, _TILE), jnp.float32),
            pltpu.VMEM((_G, d8, _SUB, _TILE), jnp.float32),
            pltpu.VMEM((d8, _SUB, tile_cols), jnp.float32),
            pltpu.SemaphoreType.DMA,
            pltpu.SemaphoreType.DMA,
        ],
    )
    def lookup(ent_hbm, eid_hbm, ent_out, eidx_v, stag_a, stag_b, eob,
               sem_a, sem_b):
        wid = lax.axis_index("s") * nc + lax.axis_index("c")
        base = wid * b_per_w
        iota = lax.iota(jnp.int32, _LANE)

        pltpu.sync_copy(eid_hbm.at[pl.ds(base, b_per_w)], eidx_v)
        stages = (stag_a, stag_b)
        sems = (sem_a, sem_b)

        def group_ids(tb, g):
            ev16 = eidx_v[pl.ds((tb + g * _G) & ~(_LANE - 1), _LANE)]
            sel = ((g & 3) * _G) + (iota & (_G - 1))
            return jnp.take(ev16, sel, axis=0)

        widths = (16, 32, 64, 128)

        def issue(tb, g, par):
            v = group_ids(tb, g)
            stag = stages[par]
            sem = sems[par]
            for i in range(_G):
                e = v[i]
                tbase = pl.multiple_of((e >> 7) << 7, _TILE)
                le = e & (_TILE - 1)
                for w in widths:

                    @pl.when((le < w) & (le >= (w // 2 if w > 16 else 0)))
                    def _():
                        pltpu.async_copy(
                            ent_hbm.at[:, :, pl.ds(tbase, w)],
                            stag.at[i, :, :, pl.ds(_TILE - w, w)], sem)
            return v

        def drain(v, par):
            stag = stages[par]
            sem = sems[par]
            for i in range(_G):
                le = v[i] & (_TILE - 1)
                for w in widths:

                    @pl.when((le < w) & (le >= (w // 2 if w > 16 else 0)))
                    def _():
                        pltpu.make_async_copy(
                            ent_hbm.at[:, :, pl.ds(0, w)],
                            stag.at[i, :, :, pl.ds(_TILE - w, w)],
                            sem).wait()

        def extract(tb, g, v, par):
            stag = stages[par]
            lev = v & (_TILE - 1)
            wv = jnp.where(lev < 16, 16,
                           jnp.where(lev < 32, 32,
                                     jnp.where(lev < 64, 64, 128)))
            lanes = _TILE - wv + lev
            slots = iota & (_G - 1)
            cols = (g * _G) + (iota & (_G - 1))
            for h in range(d8 // _G):
                c8v = h * _G + (iota >> 2)
                for sub in range(_SUB):
                    vals = plsc.load_gather(
                        stag,
                        [slots, c8v, jnp.full((_LANE,), sub, jnp.int32),
                         lanes])
                    plsc.store_scatter(eob, [c8v,
                                             jnp.full((_LANE,), sub,
                                                      jnp.int32),
                                             cols], vals)

        def tile_body(t, _):
            tb = t * tile_cols
            v0 = issue(tb, 0, 0)

            def pair_body(p, carry):
                va = carry
                g_a = 2 * p
                vb = issue(tb, g_a + 1, 1)
                drain(va, 0)
                extract(tb, g_a, va, 0)
                vnext = lax.cond(
                    p + 1 < n_pairs,
                    lambda: issue(tb, jnp.minimum(g_a + 2, n_groups - 1), 0),
                    lambda: va)
                drain(vb, 1)
                extract(tb, g_a + 1, vb, 1)
                return vnext

            lax.fori_loop(0, n_pairs, pair_body, v0)
            pltpu.sync_copy(eob, ent_out.at[:, :, pl.ds(base + tb, tile_cols)])
            return _

        lax.fori_loop(0, n_tiles, tile_body, None)

    return lookup


def _make_relation_lookup(num_relations, embed_dim, batch):
    info = plsc.get_sparse_core_info()
    nc, ns = info.num_cores, info.num_subcores
    nw = nc * ns
    b_per_w = batch // nw

    mesh = plsc.VectorSubcoreMesh(core_axis_name="c", subcore_axis_name="s")

    @functools.partial(
        pl.kernel,
        mesh=mesh,
        compiler_params=pltpu.CompilerParams(use_tc_tiling_on_sc=False),
        out_type=jax.ShapeDtypeStruct((batch, embed_dim), jnp.float32),
        scratch_types=[
            pltpu.VMEM((b_per_w,), jnp.int32),
            pltpu.VMEM((b_per_w, embed_dim), jnp.float32),
            pltpu.SemaphoreType.DMA,
        ],
    )
    def lookup(rel_hbm, rid_hbm, rel_out, ridx_v, rrows_v, rsem):
        wid = lax.axis_index("s") * nc + lax.axis_index("c")
        base = wid * b_per_w
        pltpu.sync_copy(rid_hbm.at[pl.ds(base, b_per_w)], ridx_v)
        pltpu.async_copy(rel_hbm.at[ridx_v], rrows_v, rsem).wait()
        pltpu.sync_copy(rrows_v, rel_out.at[pl.ds(base, b_per_w)])

    return lookup


def kernel(entity_embedding, relation_embedding, entity_ids, relation_ids):
    num_entities, embed_dim = entity_embedding.shape
    num_relations = relation_embedding.shape[0]
    batch = entity_ids.shape[0]
    d8 = embed_dim // _SUB
    ent_lookup = _make_entity_lookup(num_entities, embed_dim, batch)
    rel_lookup = _make_relation_lookup(num_relations, embed_dim, batch)
    ent_t = ent_lookup(
        entity_embedding.T.reshape(d8, _SUB, num_entities),
        entity_ids.astype(jnp.int32),
    )
    rel_rows = rel_lookup(relation_embedding, relation_ids.astype(jnp.int32))
    ent_rows = ent_t.reshape(embed_dim, batch).T
    return (ent_rows, rel_rows)
